# Initial kernel scaffold; baseline (speedup 1.0000x reference)
import jax
import jax.numpy as jnp
from jax.experimental import pallas as pl


def kernel(x, pos, batch, sa0_W0, sa0_b0, sa0_W1, sa0_b1, sa0_W2, sa0_b2, sa1_W0, sa1_b0, sa1_W1, sa1_b1, sa1_W2, sa1_b2, up0_W0, up0_W1, up0_W2, up1_W0, up1_W1, up1_W2, reg_W0, reg_W1, reg_W2):
    raise NotImplementedError("write your pallas kernel here")


# speedup vs baseline: 4.1029x; 4.1029x over previous
import jax
import jax.numpy as jnp
from jax.experimental import pallas as pl

K_NEIGH = 32
NEG = -1e30


def _fps_body(px_ref, py_ref, pz_ref, cx_ref, cy_ref, cz_ref, *, n_real, n_samp):
    R, C = px_ref.shape
    idx2 = (jax.lax.broadcasted_iota(jnp.int32, (R, C), 0) * C
            + jax.lax.broadcasted_iota(jnp.int32, (R, C), 1))
    pxv = px_ref[...]
    pyv = py_ref[...]
    pzv = pz_ref[...]
    pad = idx2 >= n_real
    Rs, Cs = cx_ref.shape
    cidx = (jax.lax.broadcasted_iota(jnp.int32, (Rs, Cs), 0) * Cs
            + jax.lax.broadcasted_iota(jnp.int32, (Rs, Cs), 1))

    sel0 = idx2 == 0
    lx = jnp.max(jnp.where(sel0, pxv, -jnp.inf))
    ly = jnp.max(jnp.where(sel0, pyv, -jnp.inf))
    lz = jnp.max(jnp.where(sel0, pzv, -jnp.inf))
    zero_c = jnp.zeros((Rs, Cs), jnp.float32)
    cx = jnp.where(cidx == 0, lx, zero_c)
    cy = jnp.where(cidx == 0, ly, zero_c)
    cz = jnp.where(cidx == 0, lz, zero_c)
    dmin = jnp.where(pad, -1.0, jnp.full((R, C), jnp.inf, jnp.float32))

    def step(i, st):
        dmin, lx, ly, lz, cx, cy, cz = st
        d = (pxv - lx) ** 2 + (pyv - ly) ** 2 + (pzv - lz) ** 2
        dmin = jnp.minimum(dmin, d)
        m = jnp.max(dmin)
        j = jnp.min(jnp.where(dmin == m, idx2, jnp.int32(R * C)))
        sel = idx2 == j
        lx = jnp.max(jnp.where(sel, pxv, -jnp.inf))
        ly = jnp.max(jnp.where(sel, pyv, -jnp.inf))
        lz = jnp.max(jnp.where(sel, pzv, -jnp.inf))
        put = cidx == i
        cx = jnp.where(put, lx, cx)
        cy = jnp.where(put, ly, cy)
        cz = jnp.where(put, lz, cz)
        return (dmin, lx, ly, lz, cx, cy, cz)

    st = jax.lax.fori_loop(1, n_samp, step,
                           (dmin, lx, ly, lz, cx, cy, cz))
    _, _, _, _, cx, cy, cz = st
    cx_ref[...] = cx
    cy_ref[...] = cy
    cz_ref[...] = cz


def _fps(pos, n_samp, shape_in, shape_out):
    n_real = pos.shape[0]
    R, C = shape_in
    pad = R * C - n_real
    cols = [jnp.pad(pos[:, k], (0, pad)).reshape(R, C) for k in range(3)]
    return _fps_from_cols(cols, n_real, n_samp, shape_out)


def _fps_from_cols(cols, n_real, n_samp, shape_out):
    Rs, Cs = shape_out
    import functools
    body = functools.partial(_fps_body, n_real=n_real, n_samp=n_samp)
    outs = pl.pallas_call(
        body,
        out_shape=[jax.ShapeDtypeStruct((Rs, Cs), jnp.float32)] * 3,
    )(*cols)
    cpos = jnp.stack([o.reshape(-1)[:n_samp] for o in outs], axis=-1)
    return cpos, outs


def _mlp_seq(h, Ws, bs):
    n = len(Ws)
    for i in range(n):
        h = h @ Ws[i]
        if bs[i] is not None:
            h = h + bs[i]
        if i < n - 1:
            h = jax.nn.relu(h)
    return h


def _set_abstraction(x, pos, cpos, r, Ws, bs):
    d2 = jnp.sum((cpos[:, None, :] - pos[None, :, :]) ** 2, axis=-1)
    negd, nbr = jax.lax.top_k(-d2, K_NEIGH)
    valid = (-negd) <= r * r
    msg = jnp.concatenate([x[nbr], pos[nbr] - cpos[:, None, :]], axis=-1)
    m = _mlp_seq(msg, Ws, bs)
    m = jnp.where(valid[:, :, None], m, NEG)
    out = jnp.max(m, axis=1)
    out = jnp.where(jnp.any(valid, axis=1)[:, None], out, 0.0)
    return out


def _knn_interpolate(x_src, pos_src, pos_dst, k):
    d2 = jnp.sum((pos_dst[:, None, :] - pos_src[None, :, :]) ** 2, axis=-1)
    _, idx = jax.lax.top_k(-d2, k)
    diff = pos_dst[:, None, :] - pos_src[idx]
    w = 1.0 / jnp.clip(jnp.sum(diff * diff, axis=-1), 1e-16, None)
    return jnp.sum(w[:, :, None] * x_src[idx], axis=1) / jnp.sum(w, axis=1, keepdims=True)


def kernel(x, pos, batch, sa0_W0, sa0_b0, sa0_W1, sa0_b1, sa0_W2, sa0_b2,
           sa1_W0, sa1_b0, sa1_W1, sa1_b1, sa1_W2, sa1_b2,
           up0_W0, up0_W1, up0_W2, up1_W0, up1_W1, up1_W2,
           reg_W0, reg_W1, reg_W2):
    N = pos.shape[0]
    n0 = int(round(N * 0.5))
    n1 = int(round(n0 * 0.25))

    pos0, cols0 = _fps(pos, n0, (8, -(-N // 1024) * 128), (8, 640))
    pos1, _ = _fps_from_cols(cols0, n0, n1, (8, 160))

    x0 = _set_abstraction(x, pos, pos0, 0.2,
                          [sa0_W0, sa0_W1, sa0_W2], [sa0_b0, sa0_b1, sa0_b2])
    x1 = _set_abstraction(x0, pos0, pos1, 0.4,
                          [sa1_W0, sa1_W1, sa1_W2], [sa1_b0, sa1_b1, sa1_b2])

    x1i = _knn_interpolate(x1, pos1, pos0, 3)
    h = jnp.concatenate([x1i, x0, jnp.zeros((x0.shape[0], 3), x.dtype)], axis=1)
    x2 = _mlp_seq(h, [up0_W0, up0_W1, up0_W2], [None, None, None])
    x2i = _knn_interpolate(x2, pos0, pos, 3)
    h2 = jnp.concatenate([x2i, x, jnp.zeros((x.shape[0], 3), x.dtype)], axis=1)
    x3 = _mlp_seq(h2, [up1_W0, up1_W1, up1_W2], [None, None, None])
    return _mlp_seq(x3, [reg_W0, reg_W1, reg_W2], [None, None, None])


# speedup vs baseline: 4.7612x; 1.1605x over previous
import jax
import jax.numpy as jnp
from jax.experimental import pallas as pl

K_NEIGH = 32
NEG = -1e30


def _fps_body(px_ref, py_ref, pz_ref, cx_ref, cy_ref, cz_ref, *, n_real, n_samp):
    R, C = px_ref.shape
    idx2 = (jax.lax.broadcasted_iota(jnp.int32, (R, C), 0) * C
            + jax.lax.broadcasted_iota(jnp.int32, (R, C), 1))
    pxv = px_ref[...]
    pyv = py_ref[...]
    pzv = pz_ref[...]
    pad = idx2 >= n_real
    Rs, Cs = cx_ref.shape
    cidx = (jax.lax.broadcasted_iota(jnp.int32, (Rs, Cs), 0) * Cs
            + jax.lax.broadcasted_iota(jnp.int32, (Rs, Cs), 1))

    sel0 = idx2 == 0
    lx = jnp.max(jnp.where(sel0, pxv, -jnp.inf))
    ly = jnp.max(jnp.where(sel0, pyv, -jnp.inf))
    lz = jnp.max(jnp.where(sel0, pzv, -jnp.inf))
    zero_c = jnp.zeros((Rs, Cs), jnp.float32)
    cx = jnp.where(cidx == 0, lx, zero_c)
    cy = jnp.where(cidx == 0, ly, zero_c)
    cz = jnp.where(cidx == 0, lz, zero_c)
    dmin = jnp.where(pad, -1.0, jnp.full((R, C), jnp.inf, jnp.float32))

    def step(i, st):
        dmin, lx, ly, lz, cx, cy, cz = st
        d = (pxv - lx) ** 2 + (pyv - ly) ** 2 + (pzv - lz) ** 2
        dmin = jnp.minimum(dmin, d)
        m = jnp.max(dmin)
        j = jnp.min(jnp.where(dmin == m, idx2, jnp.int32(R * C)))
        sel = idx2 == j
        lx = jnp.max(jnp.where(sel, pxv, -jnp.inf))
        ly = jnp.max(jnp.where(sel, pyv, -jnp.inf))
        lz = jnp.max(jnp.where(sel, pzv, -jnp.inf))
        put = cidx == i
        cx = jnp.where(put, lx, cx)
        cy = jnp.where(put, ly, cy)
        cz = jnp.where(put, lz, cz)
        return (dmin, lx, ly, lz, cx, cy, cz)

    st = jax.lax.fori_loop(1, n_samp, step,
                           (dmin, lx, ly, lz, cx, cy, cz))
    _, _, _, _, cx, cy, cz = st
    cx_ref[...] = cx
    cy_ref[...] = cy
    cz_ref[...] = cz


def _fps(pos, n_samp, shape_in, shape_out):
    n_real = pos.shape[0]
    R, C = shape_in
    pad = R * C - n_real
    cols = [jnp.pad(pos[:, k], (0, pad)).reshape(R, C) for k in range(3)]
    return _fps_from_cols(cols, n_real, n_samp, shape_out)


def _fps_from_cols(cols, n_real, n_samp, shape_out):
    Rs, Cs = shape_out
    import functools
    body = functools.partial(_fps_body, n_real=n_real, n_samp=n_samp)
    outs = pl.pallas_call(
        body,
        out_shape=[jax.ShapeDtypeStruct((Rs, Cs), jnp.float32)] * 3,
    )(*cols)
    cpos = jnp.stack([o.reshape(-1)[:n_samp] for o in outs], axis=-1)
    return cpos, outs


def _top3_weights(dst_ref, srcT_ref, n_src):
    Bm = dst_ref.shape[0]
    NsP = srcT_ref.shape[1]
    dx = dst_ref[:, 0:1]
    dy = dst_ref[:, 1:2]
    dz = dst_ref[:, 2:3]
    sx = srcT_ref[0:1, :]
    sy = srcT_ref[1:2, :]
    sz = srcT_ref[2:3, :]
    d2 = (dx - sx) ** 2 + (dy - sy) ** 2 + (dz - sz) ** 2
    lidx = jax.lax.broadcasted_iota(jnp.int32, (Bm, NsP), 1)
    d2 = jnp.where(lidx >= n_src, 1e30, d2)
    W = jnp.zeros((Bm, NsP), jnp.float32)
    wsum = jnp.zeros((Bm, 1), jnp.float32)
    for _ in range(3):
        m = jnp.min(d2, axis=1, keepdims=True)
        j = jnp.min(jnp.where(d2 == m, lidx, NsP), axis=1, keepdims=True)
        sel = lidx == j
        wk = 1.0 / jnp.clip(m, 1e-16, None)
        W = jnp.where(sel, wk, W)
        wsum = wsum + wk
        d2 = jnp.where(sel, 1e30, d2)
    return W, wsum


def _interp_mlp_body(dst_ref, srcT_ref, xsrc_ref, xex_ref, *rest,
                     n_src, n_layers, relu_mask):
    w_refs = rest[:n_layers + 1]
    out_ref = rest[n_layers + 1]
    W, wsum = _top3_weights(dst_ref, srcT_ref, n_src)
    xi = jnp.dot(W, xsrc_ref[...], preferred_element_type=jnp.float32) / wsum
    h = (jnp.dot(xi, w_refs[0][...], preferred_element_type=jnp.float32)
         + jnp.dot(xex_ref[...], w_refs[1][...], preferred_element_type=jnp.float32))
    if relu_mask[0]:
        h = jnp.maximum(h, 0.0)
    for li in range(1, n_layers):
        h = jnp.dot(h, w_refs[li + 1][...], preferred_element_type=jnp.float32)
        if relu_mask[li]:
            h = jnp.maximum(h, 0.0)
    out_ref[...] = h


def _interp_mlp(pos_dst, pos_src, x_src, x_extra, Ws, relu_mask, Bm=256):
    import functools
    n_dst = pos_dst.shape[0]
    n_src = pos_src.shape[0]
    NdP = -(-n_dst // Bm) * Bm
    NsP = -(-n_src // 128) * 128
    pos_dst_p = jnp.pad(pos_dst, ((0, NdP - n_dst), (0, 0)))
    srcT = jnp.pad(pos_src, ((0, NsP - n_src), (0, 0))).T
    x_src_p = jnp.pad(x_src, ((0, NsP - n_src), (0, 0)))
    x_extra_p = jnp.pad(x_extra, ((0, NdP - n_dst), (0, 0)))
    n_layers = len(Ws) - 1
    Fout = Ws[-1].shape[1]
    grid = NdP // Bm
    body = functools.partial(_interp_mlp_body, n_src=n_src,
                             n_layers=n_layers, relu_mask=relu_mask)
    full = lambda s: pl.BlockSpec(s, lambda i: (0, 0))
    out = pl.pallas_call(
        body,
        grid=(grid,),
        in_specs=[pl.BlockSpec((Bm, 3), lambda i: (i, 0)),
                  full(srcT.shape),
                  full(x_src_p.shape),
                  pl.BlockSpec((Bm, x_extra_p.shape[1]), lambda i: (i, 0))]
                 + [full(w.shape) for w in Ws],
        out_specs=pl.BlockSpec((Bm, Fout), lambda i: (i, 0)),
        out_shape=jax.ShapeDtypeStruct((NdP, Fout), jnp.float32),
    )(pos_dst_p, srcT, x_src_p, x_extra_p, *Ws)
    return out[:n_dst]


def _mlp_seq(h, Ws, bs):
    n = len(Ws)
    for i in range(n):
        h = h @ Ws[i]
        if bs[i] is not None:
            h = h + bs[i]
        if i < n - 1:
            h = jax.nn.relu(h)
    return h


def _set_abstraction(x, pos, cpos, r, Ws, bs):
    d2 = jnp.sum((cpos[:, None, :] - pos[None, :, :]) ** 2, axis=-1)
    negd, nbr = jax.lax.top_k(-d2, K_NEIGH)
    valid = (-negd) <= r * r
    msg = jnp.concatenate([x[nbr], pos[nbr] - cpos[:, None, :]], axis=-1)
    m = _mlp_seq(msg, Ws, bs)
    m = jnp.where(valid[:, :, None], m, NEG)
    out = jnp.max(m, axis=1)
    out = jnp.where(jnp.any(valid, axis=1)[:, None], out, 0.0)
    return out


def _knn_interpolate(x_src, pos_src, pos_dst, k):
    d2 = jnp.sum((pos_dst[:, None, :] - pos_src[None, :, :]) ** 2, axis=-1)
    _, idx = jax.lax.top_k(-d2, k)
    diff = pos_dst[:, None, :] - pos_src[idx]
    w = 1.0 / jnp.clip(jnp.sum(diff * diff, axis=-1), 1e-16, None)
    return jnp.sum(w[:, :, None] * x_src[idx], axis=1) / jnp.sum(w, axis=1, keepdims=True)


def kernel(x, pos, batch, sa0_W0, sa0_b0, sa0_W1, sa0_b1, sa0_W2, sa0_b2,
           sa1_W0, sa1_b0, sa1_W1, sa1_b1, sa1_W2, sa1_b2,
           up0_W0, up0_W1, up0_W2, up1_W0, up1_W1, up1_W2,
           reg_W0, reg_W1, reg_W2):
    N = pos.shape[0]
    n0 = int(round(N * 0.5))
    n1 = int(round(n0 * 0.25))

    pos0, cols0 = _fps(pos, n0, (8, -(-N // 1024) * 128), (8, 640))
    pos1, _ = _fps_from_cols(cols0, n0, n1, (8, 160))

    x0 = _set_abstraction(x, pos, pos0, 0.2,
                          [sa0_W0, sa0_W1, sa0_W2], [sa0_b0, sa0_b1, sa0_b2])
    x1 = _set_abstraction(x0, pos0, pos1, 0.4,
                          [sa1_W0, sa1_W1, sa1_W2], [sa1_b0, sa1_b1, sa1_b2])

    x2 = _interp_mlp(pos0, pos1, x1, x0,
                     [up0_W0[:2048], up0_W0[2048:2304], up0_W1, up0_W2],
                     [True, True, False])
    reg_W2p = jnp.pad(reg_W2, ((0, 0), (0, 127)))
    out = _interp_mlp(pos, pos0, x2, x,
                      [up1_W0[:256], up1_W0[256:259], up1_W1, up1_W2,
                       reg_W0, reg_W1, reg_W2p],
                      [True, True, False, True, True, False])
    return out[:, :1]


# speedup vs baseline: 10.4511x; 2.1951x over previous
import jax
import jax.numpy as jnp
from jax.experimental import pallas as pl

K_NEIGH = 32
NEG = -1e30


def _fps_body(px_ref, py_ref, pz_ref, cx_ref, cy_ref, cz_ref, *, n_real, n_samp):
    R, C = px_ref.shape
    idx2 = (jax.lax.broadcasted_iota(jnp.int32, (R, C), 0) * C
            + jax.lax.broadcasted_iota(jnp.int32, (R, C), 1))
    pxv = px_ref[...]
    pyv = py_ref[...]
    pzv = pz_ref[...]
    pad = idx2 >= n_real
    Rs, Cs = cx_ref.shape
    cidx = (jax.lax.broadcasted_iota(jnp.int32, (Rs, Cs), 0) * Cs
            + jax.lax.broadcasted_iota(jnp.int32, (Rs, Cs), 1))

    sel0 = idx2 == 0
    lx = jnp.max(jnp.where(sel0, pxv, -jnp.inf))
    ly = jnp.max(jnp.where(sel0, pyv, -jnp.inf))
    lz = jnp.max(jnp.where(sel0, pzv, -jnp.inf))
    zero_c = jnp.zeros((Rs, Cs), jnp.float32)
    cx = jnp.where(cidx == 0, lx, zero_c)
    cy = jnp.where(cidx == 0, ly, zero_c)
    cz = jnp.where(cidx == 0, lz, zero_c)
    dmin = jnp.where(pad, -1.0, jnp.full((R, C), jnp.inf, jnp.float32))

    def step(i, st):
        dmin, lx, ly, lz, cx, cy, cz = st
        d = (pxv - lx) ** 2 + (pyv - ly) ** 2 + (pzv - lz) ** 2
        dmin = jnp.minimum(dmin, d)
        m = jnp.max(dmin)
        j = jnp.min(jnp.where(dmin == m, idx2, jnp.int32(R * C)))
        sel = idx2 == j
        lx = jnp.max(jnp.where(sel, pxv, -jnp.inf))
        ly = jnp.max(jnp.where(sel, pyv, -jnp.inf))
        lz = jnp.max(jnp.where(sel, pzv, -jnp.inf))
        put = cidx == i
        cx = jnp.where(put, lx, cx)
        cy = jnp.where(put, ly, cy)
        cz = jnp.where(put, lz, cz)
        return (dmin, lx, ly, lz, cx, cy, cz)

    st = jax.lax.fori_loop(1, n_samp, step,
                           (dmin, lx, ly, lz, cx, cy, cz))
    _, _, _, _, cx, cy, cz = st
    cx_ref[...] = cx
    cy_ref[...] = cy
    cz_ref[...] = cz


def _fps(pos, n_samp, shape_in, shape_out):
    n_real = pos.shape[0]
    R, C = shape_in
    pad = R * C - n_real
    cols = [jnp.pad(pos[:, k], (0, pad)).reshape(R, C) for k in range(3)]
    return _fps_from_cols(cols, n_real, n_samp, shape_out)


def _fps_from_cols(cols, n_real, n_samp, shape_out):
    Rs, Cs = shape_out
    import functools
    body = functools.partial(_fps_body, n_real=n_real, n_samp=n_samp)
    outs = pl.pallas_call(
        body,
        out_shape=[jax.ShapeDtypeStruct((Rs, Cs), jnp.float32)] * 3,
    )(*cols)
    cpos = jnp.stack([o.reshape(-1)[:n_samp] for o in outs], axis=-1)
    return cpos, outs


def _proj_body(x_ref, pos_ref, Wa_ref, Wb_ref, p_ref):
    p_ref[...] = (jnp.dot(x_ref[...], Wa_ref[...], preferred_element_type=jnp.float32)
                  + jnp.dot(pos_ref[...], Wb_ref[...], preferred_element_type=jnp.float32))


def _point_proj(x, pos, Wa, Wb, NP):
    n = x.shape[0]
    xp = jnp.pad(x, ((0, NP - n), (0, 0)))
    pp = jnp.pad(pos, ((0, NP - n), (0, 0)))
    return pl.pallas_call(
        _proj_body,
        out_shape=jax.ShapeDtypeStruct((NP, Wa.shape[1]), jnp.float32),
    )(xp, pp, Wa, Wb)


def _sa_body(cpos_ref, posT_ref, P_ref, W0b_ref, b0_ref, W1_ref, b1_ref,
             W2_ref, b2_ref, out_ref, *, n_src, r2, n_chunks, chunk):
    Bm = cpos_ref.shape[0]
    NP = posT_ref.shape[1]
    F3 = W2_ref.shape[1]
    cx = cpos_ref[:, 0:1]
    cy = cpos_ref[:, 1:2]
    cz = cpos_ref[:, 2:3]
    d2 = ((cx - posT_ref[0:1, :]) ** 2 + (cy - posT_ref[1:2, :]) ** 2
          + (cz - posT_ref[2:3, :]) ** 2)
    lidx = jax.lax.broadcasted_iota(jnp.int32, (Bm, NP), 1)
    d2 = jnp.where(lidx >= n_src, 1e30, d2)
    P = P_ref[...]
    cterm = b0_ref[...] - jnp.dot(cpos_ref[...], W0b_ref[...],
                                  preferred_element_type=jnp.float32)
    acc = jnp.full((Bm, F3), NEG, jnp.float32)
    anyv = None
    for _ in range(n_chunks):
        h1s = []
        valids = []
        for _ in range(chunk):
            m = jnp.min(d2, axis=1, keepdims=True)
            j = jnp.min(jnp.where(d2 == m, lidx, NP), axis=1, keepdims=True)
            sel = lidx == j
            d2 = jnp.where(sel, 1e30, d2)
            g = jnp.dot(jnp.where(sel, 1.0, 0.0), P,
                        preferred_element_type=jnp.float32)
            h1s.append(g + cterm)
            valids.append(m <= r2)
            if anyv is None:
                anyv = m <= r2
        h = jnp.maximum(jnp.concatenate(h1s, axis=0), 0.0)
        h = jnp.dot(h, W1_ref[...], preferred_element_type=jnp.float32) + b1_ref[...]
        h = jnp.maximum(h, 0.0)
        h = jnp.dot(h, W2_ref[...], preferred_element_type=jnp.float32) + b2_ref[...]
        v = jnp.concatenate(valids, axis=0)
        h = jnp.where(v, h, NEG)
        acc = jnp.maximum(acc, jnp.max(h.reshape(chunk, Bm, F3), axis=0))
    out_ref[...] = jnp.where(anyv, acc, 0.0)


def _set_abstraction_pallas(x, pos, cpos, r, W0, b0, W1, b1, W2, b2,
                            Bm, chunk):
    import functools
    n_src = pos.shape[0]
    M = cpos.shape[0]
    NP = -(-n_src // 128) * 128
    MP = -(-M // Bm) * Bm
    F = x.shape[1]
    P = _point_proj(x, pos, W0[:F], W0[F:F + 3], NP)
    posT = jnp.pad(pos, ((0, NP - n_src), (0, 0))).T
    cposp = jnp.pad(cpos, ((0, MP - M), (0, 0)))
    F3 = W2.shape[1]
    body = functools.partial(_sa_body, n_src=n_src, r2=r * r,
                             n_chunks=K_NEIGH // chunk, chunk=chunk)
    full = lambda s: pl.BlockSpec(s, lambda i: (0, 0))
    out = pl.pallas_call(
        body,
        grid=(MP // Bm,),
        in_specs=[pl.BlockSpec((Bm, 3), lambda i: (i, 0)),
                  full(posT.shape), full((NP, W0.shape[1])),
                  full((3, W0.shape[1])), full((1, b0.shape[0])),
                  full(W1.shape), full((1, b1.shape[0])),
                  full(W2.shape), full((1, b2.shape[0]))],
        out_specs=pl.BlockSpec((Bm, F3), lambda i: (i, 0)),
        out_shape=jax.ShapeDtypeStruct((MP, F3), jnp.float32),
    )(cposp, posT, P, W0[F:F + 3], b0.reshape(1, -1),
      W1, b1.reshape(1, -1), W2, b2.reshape(1, -1))
    return out[:M]


def _top3_weights(dst_ref, srcT_ref, n_src):
    Bm = dst_ref.shape[0]
    NsP = srcT_ref.shape[1]
    dx = dst_ref[:, 0:1]
    dy = dst_ref[:, 1:2]
    dz = dst_ref[:, 2:3]
    sx = srcT_ref[0:1, :]
    sy = srcT_ref[1:2, :]
    sz = srcT_ref[2:3, :]
    d2 = (dx - sx) ** 2 + (dy - sy) ** 2 + (dz - sz) ** 2
    lidx = jax.lax.broadcasted_iota(jnp.int32, (Bm, NsP), 1)
    d2 = jnp.where(lidx >= n_src, 1e30, d2)
    W = jnp.zeros((Bm, NsP), jnp.float32)
    wsum = jnp.zeros((Bm, 1), jnp.float32)
    for _ in range(3):
        m = jnp.min(d2, axis=1, keepdims=True)
        j = jnp.min(jnp.where(d2 == m, lidx, NsP), axis=1, keepdims=True)
        sel = lidx == j
        wk = 1.0 / jnp.clip(m, 1e-16, None)
        W = jnp.where(sel, wk, W)
        wsum = wsum + wk
        d2 = jnp.where(sel, 1e30, d2)
    return W, wsum


def _interp_mlp_body(dst_ref, srcT_ref, xsrc_ref, xex_ref, *rest,
                     n_src, n_layers, relu_mask):
    w_refs = rest[:n_layers + 1]
    out_ref = rest[n_layers + 1]
    W, wsum = _top3_weights(dst_ref, srcT_ref, n_src)
    xi = jnp.dot(W, xsrc_ref[...], preferred_element_type=jnp.float32) / wsum
    h = (jnp.dot(xi, w_refs[0][...], preferred_element_type=jnp.float32)
         + jnp.dot(xex_ref[...], w_refs[1][...], preferred_element_type=jnp.float32))
    if relu_mask[0]:
        h = jnp.maximum(h, 0.0)
    for li in range(1, n_layers):
        h = jnp.dot(h, w_refs[li + 1][...], preferred_element_type=jnp.float32)
        if relu_mask[li]:
            h = jnp.maximum(h, 0.0)
    out_ref[...] = h


def _interp_mlp(pos_dst, pos_src, x_src, x_extra, Ws, relu_mask, Bm=256):
    import functools
    n_dst = pos_dst.shape[0]
    n_src = pos_src.shape[0]
    NdP = -(-n_dst // Bm) * Bm
    NsP = -(-n_src // 128) * 128
    pos_dst_p = jnp.pad(pos_dst, ((0, NdP - n_dst), (0, 0)))
    srcT = jnp.pad(pos_src, ((0, NsP - n_src), (0, 0))).T
    x_src_p = jnp.pad(x_src, ((0, NsP - n_src), (0, 0)))
    x_extra_p = jnp.pad(x_extra, ((0, NdP - n_dst), (0, 0)))
    n_layers = len(Ws) - 1
    Fout = Ws[-1].shape[1]
    grid = NdP // Bm
    body = functools.partial(_interp_mlp_body, n_src=n_src,
                             n_layers=n_layers, relu_mask=relu_mask)
    full = lambda s: pl.BlockSpec(s, lambda i: (0, 0))
    out = pl.pallas_call(
        body,
        grid=(grid,),
        in_specs=[pl.BlockSpec((Bm, 3), lambda i: (i, 0)),
                  full(srcT.shape),
                  full(x_src_p.shape),
                  pl.BlockSpec((Bm, x_extra_p.shape[1]), lambda i: (i, 0))]
                 + [full(w.shape) for w in Ws],
        out_specs=pl.BlockSpec((Bm, Fout), lambda i: (i, 0)),
        out_shape=jax.ShapeDtypeStruct((NdP, Fout), jnp.float32),
    )(pos_dst_p, srcT, x_src_p, x_extra_p, *Ws)
    return out[:n_dst]


def kernel(x, pos, batch, sa0_W0, sa0_b0, sa0_W1, sa0_b1, sa0_W2, sa0_b2,
           sa1_W0, sa1_b0, sa1_W1, sa1_b1, sa1_W2, sa1_b2,
           up0_W0, up0_W1, up0_W2, up1_W0, up1_W1, up1_W2,
           reg_W0, reg_W1, reg_W2):
    N = pos.shape[0]
    n0 = int(round(N * 0.5))
    n1 = int(round(n0 * 0.25))

    pos0, cols0 = _fps(pos, n0, (8, -(-N // 1024) * 128), (8, 640))
    pos1, _ = _fps_from_cols(cols0, n0, n1, (8, 160))

    x0 = _set_abstraction_pallas(x, pos, pos0, 0.2,
                                 sa0_W0, sa0_b0, sa0_W1, sa0_b1, sa0_W2, sa0_b2,
                                 Bm=256, chunk=32)
    x1 = _set_abstraction_pallas(x0, pos0, pos1, 0.4,
                                 sa1_W0, sa1_b0, sa1_W1, sa1_b1, sa1_W2, sa1_b2,
                                 Bm=128, chunk=8)

    x2 = _interp_mlp(pos0, pos1, x1, x0,
                     [up0_W0[:2048], up0_W0[2048:2304], up0_W1, up0_W2],
                     [True, True, False])
    reg_W2p = jnp.pad(reg_W2, ((0, 0), (0, 127)))
    out = _interp_mlp(pos, pos0, x2, x,
                      [up1_W0[:256], up1_W0[256:259], up1_W1, up1_W2,
                       reg_W0, reg_W1, reg_W2p],
                      [True, True, False, True, True, False])
    return out[:, :1]
